# Initial kernel scaffold; baseline (speedup 1.0000x reference)
import jax
import jax.numpy as jnp
from jax.experimental import pallas as pl


def kernel(x, weight):
    raise NotImplementedError("write your pallas kernel here")


# speedup vs baseline: 1.6843x; 1.6843x over previous
import functools

import jax
import jax.numpy as jnp
from jax import lax
from jax.experimental import pallas as pl
from jax.experimental.pallas import tpu as pltpu
from jax.experimental.pallas import tpu_sc as plsc

_VOCAB = 1_000_000
_EMB_DIM = 64
_BATCH = 16384
_HIST_LEN = 50

_NC = 2
_NS = 16
_NW = _NC * _NS

_B = _BATCH * _HIST_LEN
_B_PER_W = _B // _NW
_CHUNK = 128
_NCH = _B_PER_W // _CHUNK


def _emb_body(x_hbm, w_hbm, out_hbm, idx_v, rows_v, gsem):
    wid = lax.axis_index("s") * _NC + lax.axis_index("c")
    pltpu.sync_copy(x_hbm.at[wid], idx_v)

    def chunk(c, carry):
        pltpu.async_copy(w_hbm.at[idx_v.at[c]], rows_v, gsem).wait()
        pltpu.sync_copy(rows_v, out_hbm.at[wid, c])
        return carry

    lax.fori_loop(0, _NCH, chunk, 0)


@functools.partial(jax.jit, donate_argnums=())
def kernel(x, weight):
    xf = x.reshape(_NW, _NCH, _CHUNK)
    out = pl.kernel(
        _emb_body,
        out_type=jax.ShapeDtypeStruct((_NW, _NCH, _CHUNK, _EMB_DIM), jnp.float32),
        mesh=plsc.VectorSubcoreMesh(core_axis_name="c", subcore_axis_name="s"),
        scratch_types=[
            pltpu.VMEM((_NCH, _CHUNK), jnp.int32),
            pltpu.VMEM((_CHUNK, _EMB_DIM), jnp.float32),
            pltpu.SemaphoreType.DMA,
        ],
        compiler_params=pltpu.CompilerParams(use_tc_tiling_on_sc=False),
    )(xf, weight)
    return out.reshape(_BATCH, _HIST_LEN, _EMB_DIM)


# speedup vs baseline: 1.8746x; 1.1130x over previous
import functools

import jax
import jax.numpy as jnp
from jax import lax
from jax.experimental import pallas as pl
from jax.experimental.pallas import tpu as pltpu
from jax.experimental.pallas import tpu_sc as plsc

_VOCAB = 1_000_000
_EMB_DIM = 64
_BATCH = 16384
_HIST_LEN = 50

_NC = 2
_NS = 16
_NW = _NC * _NS

_B = _BATCH * _HIST_LEN
_B_PER_W = _B // _NW
_CHUNK = 128
_NCH = _B_PER_W // _CHUNK


_NBUF = 8


def _emb_body(x_hbm, w_hbm, out_hbm, idx_v, rows_v, *sems):
    gsems, wsems = sems[:_NBUF], sems[_NBUF:]
    wid = lax.axis_index("s") * _NC + lax.axis_index("c")
    pltpu.sync_copy(x_hbm.at[wid], idx_v)

    def gather(c, b):
        pltpu.async_copy(w_hbm.at[idx_v.at[c]], rows_v.at[b], gsems[b])

    def write(c, b):
        pltpu.async_copy(rows_v.at[b], out_hbm.at[wid, c], wsems[b])

    for b in range(_NBUF):
        gather(b, b)

    def step(c0):
        for b in range(_NBUF):
            c = c0 + b
            pltpu.make_async_copy(w_hbm.at[idx_v.at[c]], rows_v.at[b],
                                  gsems[b]).wait()
            write(c, b)
            pltpu.make_async_copy(rows_v.at[b], out_hbm.at[wid, c],
                                  wsems[b]).wait()
            gather(c + _NBUF, b)

    pl.loop(0, _NCH - _NBUF, step=_NBUF)(step)

    for b in range(_NBUF):
        c = _NCH - _NBUF + b
        pltpu.make_async_copy(w_hbm.at[idx_v.at[c]], rows_v.at[b],
                              gsems[b]).wait()
        write(c, b)
    for b in range(_NBUF):
        c = _NCH - _NBUF + b
        pltpu.make_async_copy(rows_v.at[b], out_hbm.at[wid, c],
                              wsems[b]).wait()


@functools.partial(jax.jit, donate_argnums=())
def kernel(x, weight):
    xf = x.reshape(_NW, _NCH, _CHUNK)
    out = pl.kernel(
        _emb_body,
        out_type=jax.ShapeDtypeStruct((_NW, _NCH, _CHUNK, _EMB_DIM), jnp.float32),
        mesh=plsc.VectorSubcoreMesh(core_axis_name="c", subcore_axis_name="s"),
        scratch_types=[
            pltpu.VMEM((_NCH, _CHUNK), jnp.int32),
            pltpu.VMEM((_NBUF, _CHUNK, _EMB_DIM), jnp.float32),
        ] + [pltpu.SemaphoreType.DMA] * (2 * _NBUF),
        compiler_params=pltpu.CompilerParams(use_tc_tiling_on_sc=False),
    )(xf, weight)
    return out.reshape(_BATCH, _HIST_LEN, _EMB_DIM)


# speedup vs baseline: 1.8759x; 1.0007x over previous
import functools

import jax
import jax.numpy as jnp
from jax import lax
from jax.experimental import pallas as pl
from jax.experimental.pallas import tpu as pltpu
from jax.experimental.pallas import tpu_sc as plsc

_VOCAB = 1_000_000
_EMB_DIM = 64
_BATCH = 16384
_HIST_LEN = 50

_NC = 2
_NS = 16
_NW = _NC * _NS

_B = _BATCH * _HIST_LEN
_B_PER_W = _B // _NW
_CHUNK = 128
_NCH = _B_PER_W // _CHUNK


_NBUF = 8
_LEAD = 4


def _emb_body(x_hbm, w_hbm, out_hbm, idx_v, rows_v, *sems):
    gsems, wsems = sems[:_NBUF], sems[_NBUF:]
    wid = lax.axis_index("s") * _NC + lax.axis_index("c")
    pltpu.sync_copy(x_hbm.at[wid], idx_v)

    def gather(c, b):
        pltpu.async_copy(w_hbm.at[idx_v.at[c]], rows_v.at[b], gsems[b])

    def write(c, b):
        pltpu.async_copy(rows_v.at[b], out_hbm.at[wid, c], wsems[b])

    def wait_gather(c, b):
        pltpu.make_async_copy(w_hbm.at[idx_v.at[c]], rows_v.at[b],
                              gsems[b]).wait()

    def wait_write(c, b):
        pltpu.make_async_copy(rows_v.at[b], out_hbm.at[wid, c],
                              wsems[b]).wait()

    for b in range(_LEAD):
        gather(b, b)

    for b in range(_NBUF):
        bn = (b + _LEAD) % _NBUF
        if b >= _LEAD:
            wait_write(b - _LEAD, bn)
        gather(b + _LEAD, bn)
        wait_gather(b, b)
        write(b, b)

    def step(c0):
        for b in range(_NBUF):
            c = c0 + b
            bn = (b + _LEAD) % _NBUF
            wait_write(c - _LEAD, bn)
            gather(c + _LEAD, bn)
            wait_gather(c, b)
            write(c, b)

    pl.loop(_NBUF, _NCH - _NBUF, step=_NBUF)(step)

    for b in range(_NBUF):
        c = _NCH - _NBUF + b
        bn = (b + _LEAD) % _NBUF
        wait_write(c - _LEAD, bn)
        if b < _NBUF - _LEAD:
            gather(c + _LEAD, bn)
        wait_gather(c, b)
        write(c, b)
    for b in range(_LEAD):
        wait_write(_NCH - _LEAD + b, _NBUF - _LEAD + b)


@functools.partial(jax.jit, donate_argnums=())
def kernel(x, weight):
    xf = x.reshape(_NW, _NCH, _CHUNK)
    out = pl.kernel(
        _emb_body,
        out_type=jax.ShapeDtypeStruct((_NW, _NCH, _CHUNK, _EMB_DIM), jnp.float32),
        mesh=plsc.VectorSubcoreMesh(core_axis_name="c", subcore_axis_name="s"),
        scratch_types=[
            pltpu.VMEM((_NCH, _CHUNK), jnp.int32),
            pltpu.VMEM((_NBUF, _CHUNK, _EMB_DIM), jnp.float32),
        ] + [pltpu.SemaphoreType.DMA] * (2 * _NBUF),
        compiler_params=pltpu.CompilerParams(use_tc_tiling_on_sc=False),
    )(xf, weight)
    return out.reshape(_BATCH, _HIST_LEN, _EMB_DIM)
